# baseline (speedup 1.0000x reference)
import functools

import jax
import jax.numpy as jnp
from jax import lax
from jax.experimental import pallas as pl
from jax.experimental.pallas import tpu as pltpu
from jax.experimental.pallas import tpu_sc as plsc

N_ROWS = 262144
N_UPD = 16384
D = 64
NC = 2
NS = 16
LANES = 16
CH = 16384
N_CHUNK = N_ROWS // CH
CHUNKS_PER_CORE = N_CHUNK // NC
UPD_PER_TILE = N_UPD // NS
NVEC = UPD_PER_TILE // LANES
GROUP = 128
MAX_G = UPD_PER_TILE // GROUP
ROWS_PER_TILE = CH // NS
LIST_CAP = UPD_PER_TILE + GROUP
N_PAD = 8
POS_BITS = 15
POS_MASK = (1 << POS_BITS) - 1


def _sc_body(idx_hbm, a_hbm, b_hbm, out_hbm,
             idx_v, list_f, gidx_l, gidx_p, stage, chunk, sem):
    c = lax.axis_index("c")
    s = lax.axis_index("s")
    tbase = s * UPD_PER_TILE
    lanes = lax.iota(jnp.int32, LANES)
    dummy_vec = ((lanes & 7) << POS_BITS) | (N_UPD + (lanes & 7))
    pltpu.sync_copy(idx_hbm.at[pl.ds(tbase, UPD_PER_TILE)], idx_v)

    def chunk_body(k, carry):
        base = (c * CHUNKS_PER_CORE + k) * CH

        pltpu.sync_copy(
            a_hbm.at[pl.ds(base + s * ROWS_PER_TILE, ROWS_PER_TILE)],
            chunk.at[pl.ds(s * ROWS_PER_TILE, ROWS_PER_TILE)])
        plsc.subcore_barrier()

        def scan(v, cnt):
            iv = idx_v[pl.ds(v * LANES, LANES)]
            m = (iv >= base) & (iv < base + CH)
            packed = ((iv - base) << POS_BITS) | (tbase + v * LANES + lanes)
            p = jnp.where(m, jnp.int32(1), jnp.int32(0))
            for sh in (1, 2, 4, 8):
                moved = p[jnp.maximum(lanes - sh, 0)]
                p = p + jnp.where(lanes >= sh, moved, 0)
            h = p[15]
            src = jnp.zeros((LANES,), jnp.int32)
            for sh in (8, 4, 2, 1):
                t = src + sh
                pv = p[jnp.minimum(t - 1, 15)]
                src = jnp.where(pv < lanes + 1, t, src)
            comp = packed[jnp.minimum(src, 15)]
            comp = jnp.where(lanes < h, comp, dummy_vec)
            list_f[pl.ds(cnt, LANES)] = comp
            return cnt + h
        cnt = lax.fori_loop(0, NVEC, scan, jnp.int32(0))

        for t in range(MAX_G):
            list_f[pl.ds(cnt + t * LANES, LANES)] = dummy_vec

        ng = (cnt + GROUP - 1) // GROUP
        for g in range(MAX_G):
            @pl.when(g < ng)
            def _go():
                def cp_inner(kk, _):
                    v = list_f[pl.ds(g * GROUP + kk * LANES, LANES)]
                    gidx_l[pl.ds(kk * LANES, LANES)] = (v >> POS_BITS) & 16383
                    gidx_p[pl.ds(kk * LANES, LANES)] = jnp.minimum(v & POS_MASK, N_UPD + N_PAD - 1)
                    return 0
                lax.fori_loop(0, GROUP // LANES, cp_inner, 0)
                pltpu.async_copy(b_hbm.at[gidx_p], stage, sem).wait()

                pltpu.async_copy(stage, chunk.at[gidx_l], sem, add=True).wait()
        plsc.subcore_barrier()

        pltpu.sync_copy(
            chunk.at[pl.ds(s * ROWS_PER_TILE, ROWS_PER_TILE)],
            out_hbm.at[pl.ds(base + s * ROWS_PER_TILE, ROWS_PER_TILE)])
        return 0

    lax.fori_loop(0, CHUNKS_PER_CORE, chunk_body, 0)


_scatter_add = functools.partial(
    pl.kernel,
    out_type=jax.ShapeDtypeStruct((N_ROWS, D), jnp.float32),
    mesh=plsc.VectorSubcoreMesh(core_axis_name="c", subcore_axis_name="s"),
    compiler_params=pltpu.CompilerParams(use_tc_tiling_on_sc=False),
    scratch_types=[
        pltpu.VMEM((UPD_PER_TILE,), jnp.int32),
        pltpu.VMEM((LIST_CAP,), jnp.int32),
        pltpu.VMEM((GROUP,), jnp.int32),
        pltpu.VMEM((GROUP,), jnp.int32),
        pltpu.VMEM((GROUP, D), jnp.float32),
        pltpu.VMEM_SHARED((CH + 8, D), jnp.float32),
        pltpu.SemaphoreType.DMA,
    ],
)(_sc_body)


def kernel(index, A, B):
    b_ext = jnp.concatenate([B, jnp.zeros((N_PAD, D), B.dtype)], axis=0)
    return _scatter_add(index.astype(jnp.int32), A, b_ext)
